# Initial kernel scaffold; baseline (speedup 1.0000x reference)
import jax
import jax.numpy as jnp
from jax.experimental import pallas as pl


def kernel(x, edge_index, W1, b1, g1, be1, W2, b2, g2, be2, W3, b3, g3, be3, Wj, bj):
    raise NotImplementedError("write your pallas kernel here")


# speedup vs baseline: 7.6015x; 7.6015x over previous
import functools

import jax
import jax.numpy as jnp
from jax import lax
from jax.experimental import pallas as pl
from jax.experimental.pallas import tpu as pltpu
from jax.experimental.pallas import tpu_sc as plsc

NC = 2
NS = 16
LK = 128
NB = 25


def _mm_scale(hin, Wr, deg2):
    S, N, _ = hin.shape
    br = N // NB

    def body(hin_ref, w_ref, deg_ref, out_ref):
        deg = deg_ref[0, :, 0] + deg_ref[1, :, 0] + 1.0
        dinv = lax.rsqrt(deg)
        acc = jnp.zeros((br, 128), jnp.float32)
        for a in range(S):
            acc = acc + jnp.dot(hin_ref[a], w_ref[a],
                                preferred_element_type=jnp.float32)
        out_ref[0] = acc * dinv[:, None]

    return pl.pallas_call(
        body,
        grid=(NB, 2),
        in_specs=[
            pl.BlockSpec((S, br, 128), lambda i, c: (0, i, 0)),
            pl.BlockSpec((S, 128, 128), lambda i, c: (0, 0, c)),
            pl.BlockSpec((2, br, 16), lambda i, c: (0, i, 0)),
        ],
        out_specs=pl.BlockSpec((1, br, 128), lambda i, c: (c, i, 0)),
        out_shape=jax.ShapeDtypeStruct((2, N, 128), jnp.float32),
    )(hin, Wr, deg2)


def _stats(agg2, hs2, deg2):
    _, N, _ = hs2.shape
    br = N // NB

    def body(agg_ref, hs_ref, deg_ref, t_ref, m_ref):
        i = pl.program_id(1)
        deg = deg_ref[0, :, 0] + deg_ref[1, :, 0] + 1.0
        dinv = lax.rsqrt(deg)
        t = (agg_ref[0] + hs_ref[0]) * dinv[:, None]
        t_ref[0] = t

        @pl.when(i == 0)
        def _():
            m_ref[...] = jnp.zeros_like(m_ref)

        m_ref[0, 0, :] += jnp.sum(t, axis=0)
        m_ref[0, 1, :] += jnp.sum(t * t, axis=0)

    return pl.pallas_call(
        body,
        grid=(2, NB),
        in_specs=[
            pl.BlockSpec((1, br, 128), lambda c, i: (c, i, 0)),
            pl.BlockSpec((1, br, 128), lambda c, i: (c, i, 0)),
            pl.BlockSpec((2, br, 16), lambda c, i: (0, i, 0)),
        ],
        out_specs=[
            pl.BlockSpec((1, br, 128), lambda c, i: (c, i, 0)),
            pl.BlockSpec((1, 2, 128), lambda c, i: (c, 0, 0)),
        ],
        out_shape=[
            jax.ShapeDtypeStruct((2, N, 128), jnp.float32),
            jax.ShapeDtypeStruct((2, 2, 128), jnp.float32),
        ],
    )(agg2, hs2, deg2)


def _bnrelu(t2, m2, gr, ber, eps=1e-5):
    _, N, _ = t2.shape
    br = N // NB
    n_f = float(N)

    def body(t_ref, m_ref, g_ref, be_ref, y_ref):
        mu = m_ref[0, 0, :] / n_f
        var = m_ref[0, 1, :] / n_f - mu * mu
        scale = lax.rsqrt(var + eps) * g_ref[0, 0, :]
        y = (t_ref[0] - mu[None, :]) * scale[None, :] + be_ref[0, 0, :][None, :]
        y_ref[0] = jnp.maximum(y, 0.0)

    return pl.pallas_call(
        body,
        grid=(2, NB),
        in_specs=[
            pl.BlockSpec((1, br, 128), lambda c, i: (c, i, 0)),
            pl.BlockSpec((1, 2, 128), lambda c, i: (c, 0, 0)),
            pl.BlockSpec((1, 1, 128), lambda c, i: (c, 0, 0)),
            pl.BlockSpec((1, 1, 128), lambda c, i: (c, 0, 0)),
        ],
        out_specs=pl.BlockSpec((1, br, 128), lambda c, i: (c, i, 0)),
        out_shape=jax.ShapeDtypeStruct((2, N, 128), jnp.float32),
    )(t2, m2, gr, ber)


def _head(y1, y2, y3, Wjr, bjr):
    _, N, _ = y1.shape
    C = Wjr.shape[3]
    br = N // NB

    def body(y1_ref, y2_ref, y3_ref, w_ref, b_ref, out_ref):
        acc = jnp.broadcast_to(b_ref[...], (br, C)).astype(jnp.float32)
        for l, yr in enumerate((y1_ref, y2_ref, y3_ref)):
            for a in range(2):
                acc = acc + jnp.dot(yr[a], w_ref[l, a],
                                    preferred_element_type=jnp.float32)
        m = jnp.max(acc, axis=1, keepdims=True)
        z = acc - m
        lse = jnp.log(jnp.sum(jnp.exp(z), axis=1, keepdims=True))
        out_ref[...] = z - lse

    yspec = pl.BlockSpec((2, br, 128), lambda i: (0, i, 0))
    return pl.pallas_call(
        body,
        grid=(NB,),
        in_specs=[
            yspec, yspec, yspec,
            pl.BlockSpec((3, 2, 128, C), lambda i: (0, 0, 0, 0)),
            pl.BlockSpec((1, C), lambda i: (0, 0)),
        ],
        out_specs=pl.BlockSpec((br, C), lambda i: (i, 0)),
        out_shape=jax.ShapeDtypeStruct((N, C), jnp.float32),
    )(y1, y2, y3, Wjr, bjr)


def _deg_sc(dstd, ones_rows, zrows, npad, td):
    rpt = npad // NS
    mesh = plsc.VectorSubcoreMesh(core_axis_name="c", subcore_axis_name="s")

    @functools.partial(
        pl.kernel,
        out_type=jax.ShapeDtypeStruct((2, npad, 16), jnp.float32),
        mesh=mesh,
        scratch_types=[
            pltpu.VMEM((td, LK), jnp.int32),
            pltpu.VMEM((LK, 16), jnp.float32),
            pltpu.VMEM_SHARED((npad, 16), jnp.float32),
        ],
    )
    def k(dstd_hbm, ones_hbm, z_hbm, out_hbm, dst_v, ones_v, acc):
        c = lax.axis_index("c")
        s = lax.axis_index("s")
        pltpu.sync_copy(z_hbm, acc.at[pl.ds(s * rpt, rpt)])
        pltpu.sync_copy(dstd_hbm.at[c, s], dst_v)
        pltpu.sync_copy(ones_hbm, ones_v)
        plsc.subcore_barrier()

        def body(j, carry):
            pltpu.sync_copy(ones_v, acc.at[dst_v.at[j]], add=True)
            return carry

        lax.fori_loop(0, td, body, 0)
        plsc.subcore_barrier()
        pltpu.sync_copy(acc.at[pl.ds(s * rpt, rpt)],
                        out_hbm.at[c].at[pl.ds(s * rpt, rpt)])

    return k(dstd, ones_rows, zrows)


SB = 16


def _agg_sc(hs_flat, srcg, dstg, zrows, npad, t):
    rpt = npad // NS
    nsb = t // SB
    mesh = plsc.VectorSubcoreMesh(core_axis_name="c", subcore_axis_name="s")

    @functools.partial(
        pl.kernel,
        out_type=jax.ShapeDtypeStruct((2, npad, 128), jnp.float32),
        mesh=mesh,
        scratch_types=[
            pltpu.VMEM((SB, LK), jnp.int32),
            pltpu.VMEM((SB, LK), jnp.int32),
            pltpu.VMEM((LK, 128), jnp.float32),
            pltpu.VMEM((LK, 128), jnp.float32),
            pltpu.VMEM_SHARED((npad, 128), jnp.float32),
            pltpu.SemaphoreType.DMA,
        ],
    )
    def k(hs_hbm, srcg_hbm, dstg_hbm, z_hbm, out_hbm,
          src_v, dst_v, rows0, rows1, acc, sem):
        c = lax.axis_index("c")
        s = lax.axis_index("s")
        pltpu.sync_copy(z_hbm, acc.at[pl.ds(s * rpt, rpt)])
        plsc.subcore_barrier()

        def super_body(sb, carry):
            pltpu.sync_copy(srcg_hbm.at[c, s, pl.ds(sb * SB, SB)], src_v)
            pltpu.sync_copy(dstg_hbm.at[s, pl.ds(sb * SB, SB)], dst_v)
            pltpu.async_copy(hs_hbm.at[src_v.at[0]], rows0, sem)

            def body(i, carry2):
                j0 = 2 * i
                pltpu.make_async_copy(
                    hs_hbm.at[src_v.at[0]], rows0, sem).wait()
                pltpu.async_copy(hs_hbm.at[src_v.at[j0 + 1]], rows1, sem)
                pltpu.sync_copy(rows0, acc.at[dst_v.at[j0]], add=True)
                pltpu.make_async_copy(
                    hs_hbm.at[src_v.at[0]], rows1, sem).wait()

                @pl.when(i + 1 < SB // 2)
                def _():
                    pltpu.async_copy(hs_hbm.at[src_v.at[j0 + 2]], rows0, sem)

                pltpu.sync_copy(rows1, acc.at[dst_v.at[j0 + 1]], add=True)
                return carry2

            lax.fori_loop(0, SB // 2, body, 0)
            return carry

        lax.fori_loop(0, nsb, super_body, 0)
        plsc.subcore_barrier()
        pltpu.sync_copy(acc.at[pl.ds(s * rpt, rpt)],
                        out_hbm.at[c].at[pl.ds(s * rpt, rpt)])

    return k(hs_flat, srcg, dstg, zrows)


def kernel(x, edge_index, W1, b1, g1, be1, W2, b2, g2, be2,
           W3, b3, g3, be3, Wj, bj):
    N, F = x.shape
    H = W1.shape[1]
    C = Wj.shape[1]
    E = edge_index.shape[1]
    src = edge_index[0]
    dst = edge_index[1]
    npad = 10240

    ept = E // NS
    t = -(-ept // (LK * SB)) * SB
    padn = t * LK - ept
    srcp = jnp.concatenate(
        [src.reshape(NS, ept),
         jnp.zeros((NS, padn), jnp.int32)], axis=1)
    dstp = jnp.concatenate(
        [dst.reshape(NS, ept),
         jnp.full((NS, padn), N, jnp.int32)], axis=1)
    srcg = jnp.stack([srcp, srcp + N]).reshape(2, NS, t, LK)
    dstg = dstp.reshape(NS, t, LK)

    ept2 = E // (2 * NS)
    td = -(-ept2 // LK)
    padd = td * LK - ept2
    dstd = jnp.concatenate(
        [dst.reshape(2, NS, ept2),
         jnp.full((2, NS, padd), N, jnp.int32)], axis=2).reshape(2, NS, td, LK)

    zr_a = jnp.zeros((npad // NS, 128), jnp.float32)
    zr_d = jnp.zeros((npad // NS, 16), jnp.float32)
    ones_rows = jnp.ones((LK, 16), jnp.float32)

    deg2 = _deg_sc(dstd, ones_rows, zr_d, npad, td)

    h = x.reshape(1, N, F)
    ys = []
    for (W, g, be) in ((W1, g1, be1), (W2, g2, be2), (W3, g3, be3)):
        S = h.shape[0]
        Wr = W.reshape(S, 128, H)
        hs2 = _mm_scale(h, Wr, deg2)
        agg2 = _agg_sc(hs2.reshape(2 * N, 128), srcg, dstg, zr_a, npad, t)
        t2, m2 = _stats(agg2, hs2, deg2)
        y = _bnrelu(t2, m2, g.reshape(2, 1, 128), be.reshape(2, 1, 128))
        ys.append(y)
        h = y

    return _head(ys[0], ys[1], ys[2], Wj.reshape(3, 2, 128, C),
                 bj.reshape(1, C))
